# baseline (speedup 1.0000x reference)
import functools

import jax
import jax.numpy as jnp
from jax import lax
from jax.experimental import pallas as pl
from jax.experimental.pallas import tpu as pltpu
from jax.experimental.pallas import tpu_sc as plsc

_NQ = 4096
_NS = 128
_D = 128
_NW = 32
_QPW = _NQ // _NW
_L = 16
_EPS = 1e-8


def _hsum16(v):
    return jnp.broadcast_to(jnp.sum(v), (_L,))


def _rsqrt16(x):
    i = lax.bitcast_convert_type(x, jnp.int32)
    i = jnp.int32(0x5F3759DF) - (i >> 1)
    y = lax.bitcast_convert_type(i, jnp.float32)
    for _ in range(3):
        y = y * (1.5 - 0.5 * x * y * y)
    return y


def _body(qf_hbm, sf_hbm, table_hbm, out_hbm,
          qidx_v, sidx_v, qbuf, sbuf, part_v, allbuf,
          out_v, shared, qsem, ssem):
    cid = lax.axis_index("c")
    sid = lax.axis_index("s")
    wid = sid * 2 + cid
    lane = lax.iota(jnp.int32, _L)

    pltpu.sync_copy(sf_hbm.at[pl.ds(sid * _L, _L)], sidx_v)
    scp = pltpu.async_copy(table_hbm.at[sidx_v], sbuf, ssem)

    pltpu.sync_copy(qf_hbm.at[pl.ds(wid * 2 * _QPW, 2 * _QPW)], qidx_v)
    qcp0 = pltpu.async_copy(table_hbm.at[qidx_v.at[pl.ds(0, _QPW)]],
                            qbuf.at[pl.ds(0, _QPW)], qsem)
    qcp1 = pltpu.async_copy(table_hbm.at[qidx_v.at[pl.ds(_QPW, _QPW)]],
                            qbuf.at[pl.ds(_QPW, _QPW)], qsem)

    scp.wait()
    for h in range(2):
        for c in range(8):
            acc = sbuf[h, pl.ds(c * _L, _L)]
            for r in range(1, 8):
                acc = acc + sbuf[2 * r + h, pl.ds(c * _L, _L)]
            part_v[pl.ds(h * 128 + c * _L, _L)] = acc
    pltpu.sync_copy(part_v, shared.at[sid])
    plsc.subcore_barrier()
    pltpu.sync_copy(shared, allbuf)

    mv = []
    for c in range(16):
        acc = allbuf[0, pl.ds(c * _L, _L)]
        for t in range(1, 16):
            acc = acc + allbuf[t, pl.ds(c * _L, _L)]
        mv.append(acc)

    accm = mv[0] * mv[0]
    for c in range(1, 16):
        accm = accm + mv[c] * mv[c]
    nm2v = _hsum16(accm)
    nmv = nm2v * _rsqrt16(jnp.maximum(nm2v, 1e-30)) * (1.0 / _NS)
    scale_v = 1.0 / (_NS * jnp.maximum(nmv, _EPS))

    qcp0.wait()
    qcp1.wait()

    def gstep(g, carry):
        dvec = jnp.zeros((_L,), jnp.float32)
        nvec = jnp.zeros((_L,), jnp.float32)
        for j in range(_L):
            q = g * _L + j
            accd = jnp.zeros((_L,), jnp.float32)
            accn = jnp.zeros((_L,), jnp.float32)
            for h in range(2):
                for c in range(8):
                    e = qbuf[2 * q + h, pl.ds(c * _L, _L)]
                    m = mv[h * 8 + c]
                    accd = accd + e * m
                    accn = accn + e * e
            dvec = jnp.where(lane == j, _hsum16(accd), dvec)
            nvec = jnp.where(lane == j, _hsum16(accn), nvec)
        y = _rsqrt16(jnp.maximum(nvec, 1e-30))
        inv = jnp.where(nvec >= 1e-16, y, 1.0 / _EPS)
        out_v[pl.ds(g * _L, _L)] = dvec * inv * scale_v
        return carry

    lax.fori_loop(0, _QPW // _L, gstep, 0)
    pltpu.sync_copy(out_v, out_hbm.at[pl.ds(wid * _QPW, _QPW)])


@functools.partial(
    pl.kernel,
    out_type=jax.ShapeDtypeStruct((_NQ,), jnp.float32),
    mesh=plsc.VectorSubcoreMesh(core_axis_name="c", subcore_axis_name="s"),
    compiler_params=pltpu.CompilerParams(needs_layout_passes=False),
    scratch_types=[
        pltpu.VMEM((2 * _QPW,), jnp.int32),
        pltpu.VMEM((_L,), jnp.int32),
        pltpu.VMEM((2 * _QPW, _D), jnp.float32),
        pltpu.VMEM((_L, _D), jnp.float32),
        pltpu.VMEM((2 * _D,), jnp.float32),
        pltpu.VMEM((_L, 2 * _D), jnp.float32),
        pltpu.VMEM((_QPW,), jnp.float32),
        pltpu.VMEM_SHARED((_L, 2 * _D), jnp.float32),
        pltpu.SemaphoreType.DMA,
        pltpu.SemaphoreType.DMA,
    ],
)
def _sc_embed_matcher(qt_hbm, st_hbm, table_hbm, out_hbm, *scratch):
    _body(qt_hbm, st_hbm, table_hbm, out_hbm, *scratch)


def kernel(query, support, emb_table):
    if query.dtype != jnp.int32:
        query = query.astype(jnp.int32)
    if support.dtype != jnp.int32:
        support = support.astype(jnp.int32)
    return _sc_embed_matcher(query.reshape(-1), support.reshape(-1),
                             emb_table)


# speedup vs baseline: 1.0175x; 1.0175x over previous
import functools

import jax
import jax.numpy as jnp
from jax import lax
from jax.experimental import pallas as pl
from jax.experimental.pallas import tpu as pltpu
from jax.experimental.pallas import tpu_sc as plsc

_NQ = 4096
_NS = 128
_D = 128
_NW = 32
_QPW = _NQ // _NW
_H = _QPW // 2
_L = 16
_EPS = 1e-8


def _hsum16(v):
    return jnp.broadcast_to(jnp.sum(v), (_L,))


def _rsqrt16(x):
    i = lax.bitcast_convert_type(x, jnp.int32)
    i = jnp.int32(0x5F3759DF) - (i >> 1)
    y = lax.bitcast_convert_type(i, jnp.float32)
    for _ in range(3):
        y = y * (1.5 - 0.5 * x * y * y)
    return y


def _body(qt_hbm, st_hbm, table_hbm, out_hbm,
          qidx_v, sidx_v, qbuf, sbuf, part_v, allbuf,
          out_v, shared, qsemA, qsemB, ssem, isem):
    cid = lax.axis_index("c")
    sid = lax.axis_index("s")
    wid = sid * 2 + cid
    lane = lax.iota(jnp.int32, _L)

    i0 = pltpu.async_copy(st_hbm.at[0, pl.ds(sid * 8, 8)],
                          sidx_v.at[pl.ds(0, 8)], isem)
    i1 = pltpu.async_copy(st_hbm.at[1, pl.ds(sid * 8, 8)],
                          sidx_v.at[pl.ds(8, 8)], isem)
    i2 = pltpu.async_copy(qt_hbm.at[0, pl.ds(wid * _QPW, _QPW)],
                          qidx_v.at[pl.ds(0, _QPW)], isem)
    i3 = pltpu.async_copy(qt_hbm.at[1, pl.ds(wid * _QPW, _QPW)],
                          qidx_v.at[pl.ds(_QPW, _QPW)], isem)

    i0.wait()
    i1.wait()
    scp = pltpu.async_copy(table_hbm.at[sidx_v], sbuf, ssem)

    i2.wait()
    i3.wait()
    qa0 = pltpu.async_copy(table_hbm.at[qidx_v.at[pl.ds(0, _H)]],
                           qbuf.at[pl.ds(0, _H)], qsemA)
    qa1 = pltpu.async_copy(table_hbm.at[qidx_v.at[pl.ds(_QPW, _H)]],
                           qbuf.at[pl.ds(_QPW, _H)], qsemA)
    qb0 = pltpu.async_copy(table_hbm.at[qidx_v.at[pl.ds(_H, _H)]],
                           qbuf.at[pl.ds(_H, _H)], qsemB)
    qb1 = pltpu.async_copy(table_hbm.at[qidx_v.at[pl.ds(_QPW + _H, _H)]],
                           qbuf.at[pl.ds(_QPW + _H, _H)], qsemB)

    scp.wait()
    for h in range(2):
        for c in range(8):
            acc = sbuf[h * 8, pl.ds(c * _L, _L)]
            for r in range(1, 8):
                acc = acc + sbuf[h * 8 + r, pl.ds(c * _L, _L)]
            part_v[pl.ds(h * 128 + c * _L, _L)] = acc
    pltpu.sync_copy(part_v, shared.at[sid])
    plsc.subcore_barrier()
    pltpu.sync_copy(shared, allbuf)

    mv = []
    for c in range(16):
        acc = allbuf[0, pl.ds(c * _L, _L)]
        for t in range(1, 16):
            acc = acc + allbuf[t, pl.ds(c * _L, _L)]
        mv.append(acc)

    accm = mv[0] * mv[0]
    for c in range(1, 16):
        accm = accm + mv[c] * mv[c]
    nm2v = _hsum16(accm)
    nmv = nm2v * _rsqrt16(jnp.maximum(nm2v, 1e-30)) * (1.0 / _NS)
    scale_v = 1.0 / (_NS * jnp.maximum(nmv, _EPS))

    def gstep(g, carry):
        dvec = jnp.zeros((_L,), jnp.float32)
        nvec = jnp.zeros((_L,), jnp.float32)
        for j in range(_L):
            q = g * _L + j
            accd = jnp.zeros((_L,), jnp.float32)
            accn = jnp.zeros((_L,), jnp.float32)
            for h in range(2):
                for c in range(8):
                    e = qbuf[q + h * _QPW, pl.ds(c * _L, _L)]
                    m = mv[h * 8 + c]
                    accd = accd + e * m
                    accn = accn + e * e
            dvec = jnp.where(lane == j, _hsum16(accd), dvec)
            nvec = jnp.where(lane == j, _hsum16(accn), nvec)
        y = _rsqrt16(jnp.maximum(nvec, 1e-30))
        inv = jnp.where(nvec >= 1e-16, y, 1.0 / _EPS)
        out_v[pl.ds(g * _L, _L)] = dvec * inv * scale_v
        return carry

    qa0.wait()
    qa1.wait()
    lax.fori_loop(0, _H // _L, gstep, 0)
    qb0.wait()
    qb1.wait()
    lax.fori_loop(_H // _L, _QPW // _L, gstep, 0)
    pltpu.sync_copy(out_v, out_hbm.at[pl.ds(wid * _QPW, _QPW)])


@functools.partial(
    pl.kernel,
    out_type=jax.ShapeDtypeStruct((_NQ,), jnp.float32),
    mesh=plsc.VectorSubcoreMesh(core_axis_name="c", subcore_axis_name="s"),
    compiler_params=pltpu.CompilerParams(needs_layout_passes=False),
    scratch_types=[
        pltpu.VMEM((2 * _QPW,), jnp.int32),
        pltpu.VMEM((_L,), jnp.int32),
        pltpu.VMEM((2 * _QPW, _D), jnp.float32),
        pltpu.VMEM((_L, _D), jnp.float32),
        pltpu.VMEM((2 * _D,), jnp.float32),
        pltpu.VMEM((_L, 2 * _D), jnp.float32),
        pltpu.VMEM((_QPW,), jnp.float32),
        pltpu.VMEM_SHARED((_L, 2 * _D), jnp.float32),
        pltpu.SemaphoreType.DMA,
        pltpu.SemaphoreType.DMA,
        pltpu.SemaphoreType.DMA,
        pltpu.SemaphoreType.DMA,
    ],
)
def _sc_embed_matcher(qt_hbm, st_hbm, table_hbm, out_hbm, *scratch):
    _body(qt_hbm, st_hbm, table_hbm, out_hbm, *scratch)


def kernel(query, support, emb_table):
    if query.dtype != jnp.int32:
        query = query.astype(jnp.int32)
    if support.dtype != jnp.int32:
        support = support.astype(jnp.int32)
    return _sc_embed_matcher(query.T, support.T, emb_table)
